# baseline (speedup 1.0000x reference)
import functools

import jax
import jax.numpy as jnp
from jax import lax
from jax.experimental import pallas as pl
from jax.experimental.pallas import tpu as pltpu
from jax.experimental.pallas import tpu_sc as plsc

N = 10000
E = 160000
NODE_IN = 128
EDGE_IN = 16
H = 32
W = 128
OUT = 128
STEPS = 3
S2S_STEPS = 3
NGRAPH = 256

NW = 32
CH = 128
E_PAD = 163840
EPW = E_PAD // NW
NCH = EPW // CH
N_PAD = 10240
ROWS_PER_TILE = N_PAD // 16


def _sc_gather_body(h_hbm, src_hbm, out_hbm, idx_all, rows, sem):
    c = lax.axis_index("c")
    s = lax.axis_index("s")
    wid = s * 2 + c
    pltpu.sync_copy(src_hbm.at[wid], idx_all)

    def chunk(j, carry):
        off = wid * EPW + j * CH
        pltpu.async_copy(h_hbm.at[idx_all.at[j]], rows, sem).wait()
        pltpu.sync_copy(rows, out_hbm.at[pl.ds(off, CH)])
        return carry

    lax.fori_loop(0, NCH, chunk, 0)


@functools.cache
def _sc_gather_call():
    mesh = plsc.VectorSubcoreMesh(core_axis_name="c", subcore_axis_name="s")
    return pl.kernel(
        _sc_gather_body,
        out_type=jax.ShapeDtypeStruct((E_PAD, W), jnp.float32),
        mesh=mesh,
        scratch_types=[
            pltpu.VMEM((NCH, CH), jnp.int32),
            pltpu.VMEM((CH, W), jnp.float32),
            pltpu.SemaphoreType.DMA,
        ],
    )


def _sc_gather(h_wide, src3):
    return _sc_gather_call()(h_wide, src3)


def _sc_scatter_body(msg_hbm, dst_hbm, zeros_hbm, out_hbm, idx_all, rows,
                     accum, sem):
    c = lax.axis_index("c")
    s = lax.axis_index("s")
    wid = s * 2 + c

    @pl.when(s == 0)
    def _():
        pltpu.sync_copy(zeros_hbm, accum)

    plsc.subcore_barrier()
    pltpu.sync_copy(dst_hbm.at[wid], idx_all)

    def chunk(j, carry):
        off = wid * EPW + j * CH
        pltpu.sync_copy(msg_hbm.at[pl.ds(off, CH)], rows)
        pltpu.sync_copy(rows, accum.at[idx_all.at[j]], add=True)
        return carry

    lax.fori_loop(0, NCH, chunk, 0)
    plsc.subcore_barrier()
    pltpu.sync_copy(accum.at[pl.ds(s * ROWS_PER_TILE, ROWS_PER_TILE)],
                    out_hbm.at[c, pl.ds(s * ROWS_PER_TILE, ROWS_PER_TILE)])


@functools.cache
def _sc_scatter_call():
    mesh = plsc.VectorSubcoreMesh(core_axis_name="c", subcore_axis_name="s")
    return pl.kernel(
        _sc_scatter_body,
        out_type=jax.ShapeDtypeStruct((2, N_PAD, W), jnp.float32),
        mesh=mesh,
        scratch_types=[
            pltpu.VMEM((NCH, CH), jnp.int32),
            pltpu.VMEM((CH, W), jnp.float32),
            pltpu.VMEM_SHARED((N_PAD, W), jnp.float32),
            pltpu.SemaphoreType.DMA,
        ],
    )


def _sc_scatter(msg_wide, dst3, zeros_npad):
    return _sc_scatter_call()(msg_wide, dst3, zeros_npad)


def _node_mlp_body(x_ref, w_ref, b_ref, o_ref):
    h = jnp.maximum(
        jnp.dot(x_ref[...], w_ref[...], preferred_element_type=jnp.float32)
        + b_ref[...], 0.0)
    o_ref[...] = jnp.concatenate(
        [h, jnp.zeros((N, W - H), jnp.float32)], axis=1)


def _node_mlp(x, W1, b1):
    return pl.pallas_call(
        _node_mlp_body,
        out_shape=jax.ShapeDtypeStruct((N, W), jnp.float32),
    )(x, W1, b1.reshape(1, H))


EB = 1024
N_EB = E_PAD // EB


def _msg_body(ea_ref, hs_ref, we_ref, be_ref, o_ref):
    a = jnp.dot(ea_ref[...], we_ref[...], preferred_element_type=jnp.float32)
    ew = jnp.maximum(a + be_ref[...], 0.0)
    hs = hs_ref[...]
    acc = hs[:, 0:1] * ew[:, 0:H]
    for hh in range(1, H):
        acc = acc + hs[:, hh:hh + 1] * ew[:, hh * H:(hh + 1) * H]
    o_ref[...] = jnp.concatenate(
        [acc, jnp.zeros((EB, W - H), jnp.float32)], axis=1)


def _msg(ea_pad, hsrc_wide, We, be):
    return pl.pallas_call(
        _msg_body,
        grid=(N_EB,),
        in_specs=[
            pl.BlockSpec((EB, EDGE_IN), lambda i: (i, 0)),
            pl.BlockSpec((EB, W), lambda i: (i, 0)),
            pl.BlockSpec((EDGE_IN, H * H), lambda i: (0, 0)),
            pl.BlockSpec((1, H * H), lambda i: (0, 0)),
        ],
        out_specs=pl.BlockSpec((EB, W), lambda i: (i, 0)),
        out_shape=jax.ShapeDtypeStruct((E_PAD, W), jnp.float32),
        compiler_params=pltpu.CompilerParams(
            dimension_semantics=("arbitrary",)),
    )(ea_pad, hsrc_wide, We, be.reshape(1, H * H))


def _gru_body(p_ref, h_ref, root_ref, bc_ref, wih_ref, bih_ref, whh_ref,
              bhh_ref, o_ref):
    h = h_ref[:, 0:H]
    aggr = p_ref[0, :N, 0:H] + p_ref[1, :N, 0:H]
    conv = aggr + jnp.dot(h, root_ref[...],
                          preferred_element_type=jnp.float32) + bc_ref[...]
    m = jnp.maximum(conv, 0.0)
    gi = jnp.dot(m, wih_ref[...], preferred_element_type=jnp.float32) \
        + bih_ref[...]
    gh = jnp.dot(h, whh_ref[...], preferred_element_type=jnp.float32) \
        + bhh_ref[...]
    r = jax.nn.sigmoid(gi[:, 0:H] + gh[:, 0:H])
    z = jax.nn.sigmoid(gi[:, H:2 * H] + gh[:, H:2 * H])
    n_ = jnp.tanh(gi[:, 2 * H:3 * H] + r * gh[:, 2 * H:3 * H])
    hn = (1.0 - z) * n_ + z * h
    o_ref[...] = jnp.concatenate(
        [hn, jnp.zeros((N, W - H), jnp.float32)], axis=1)


def _gru(partials, h_wide, root, bconv, WihT, b_ih, WhhT, b_hh):
    return pl.pallas_call(
        _gru_body,
        out_shape=jax.ShapeDtypeStruct((N, W), jnp.float32),
    )(partials, h_wide, root, bconv.reshape(1, H), WihT,
      b_ih.reshape(1, 3 * H), WhhT, b_hh.reshape(1, 3 * H))


def _ln_prelu(y, g, b, alpha):
    mu = jnp.mean(y, axis=-1, keepdims=True)
    var = jnp.mean((y - mu) ** 2, axis=-1, keepdims=True)
    yn = (y - mu) / jnp.sqrt(var + 1e-5) * g + b
    return jnp.where(yn >= 0, yn, alpha * yn)


S2S_CHK = 1000
S2S_NCHK = N // S2S_CHK


def _s2s_body(h0_ref, h3_ref, batch_ref, sf_ref, wlih_ref, blih_ref,
              wlhh_ref, blhh_ref, wp_ref, bp_ref, g1_ref, be1_ref, a1_ref,
              wst_ref, wsb_ref, bs_ref, g2_ref, be2_ref, a2_ref, o_ref):
    def chunk(i):
        sl = pl.ds(i * S2S_CHK, S2S_CHK)
        na_c = jnp.concatenate([h0_ref[sl, 0:H], h3_ref[sl, 0:H]], axis=1)
        gid = lax.broadcasted_iota(jnp.int32, (S2S_CHK, NGRAPH), 1)
        onehot = batch_ref[sl, :] == gid
        return na_c, onehot

    q_star = jnp.zeros((NGRAPH, 4 * H), jnp.float32)
    hl = jnp.zeros((NGRAPH, 2 * H), jnp.float32)
    cl = jnp.zeros((NGRAPH, 2 * H), jnp.float32)
    for _ in range(S2S_STEPS):
        gates = (jnp.dot(q_star, wlih_ref[...],
                         preferred_element_type=jnp.float32) + blih_ref[...]
                 + jnp.dot(hl, wlhh_ref[...],
                           preferred_element_type=jnp.float32)
                 + blhh_ref[...])
        ig = jax.nn.sigmoid(gates[:, 0:2 * H])
        fg = jax.nn.sigmoid(gates[:, 2 * H:4 * H])
        gg = jnp.tanh(gates[:, 4 * H:6 * H])
        og = jax.nn.sigmoid(gates[:, 6 * H:8 * H])
        cl = fg * cl + ig * gg
        hl = og * jnp.tanh(cl)
        q = hl

        def maxpass(i, emax):
            na_c, onehot = chunk(i)
            emat = lax.dot_general(na_c, q, (((1,), (1,)), ((), ())),
                                   preferred_element_type=jnp.float32)
            m = jnp.max(jnp.where(onehot, emat, -1e30), axis=0,
                        keepdims=True)
            return jnp.maximum(emax, m)

        emax = lax.fori_loop(0, S2S_NCHK, maxpass,
                             jnp.full((1, NGRAPH), -1e30, jnp.float32))
        emax = jnp.where(emax > -1e29, emax, 0.0)

        def sumpass(i, carry):
            denom8, U = carry
            na_c, onehot = chunk(i)
            onef = onehot.astype(jnp.float32)
            emat = lax.dot_general(na_c, q, (((1,), (1,)), ((), ())),
                                   preferred_element_type=jnp.float32)
            e = jnp.sum(jnp.where(onehot, emat, 0.0), axis=1, keepdims=True)
            emaxb = jnp.sum(onef * emax, axis=1, keepdims=True)
            ex = jnp.exp(e - emaxb)
            w = onef * ex
            d8 = lax.dot_general(w, jnp.ones((S2S_CHK, 8), jnp.float32),
                                 (((0,), (0,)), ((), ())),
                                 preferred_element_type=jnp.float32)
            dU = lax.dot_general(w, na_c, (((0,), (0,)), ((), ())),
                                 preferred_element_type=jnp.float32)
            return denom8 + d8, U + dU

        denom8, U = lax.fori_loop(
            0, S2S_NCHK, sumpass,
            (jnp.zeros((NGRAPH, 8), jnp.float32),
             jnp.zeros((NGRAPH, 2 * H), jnp.float32)))
        r_ = U / (denom8[:, 0:1] + 1e-16)
        q_star = jnp.concatenate([q, r_], axis=1)

    go = _ln_prelu(jnp.dot(q_star, wp_ref[...],
                           preferred_element_type=jnp.float32) + bp_ref[...],
                   g1_ref[...], be1_ref[...], a1_ref[0, 0])
    fused_pre = (jnp.dot(go, wst_ref[...],
                         preferred_element_type=jnp.float32)
                 + jnp.dot(sf_ref[...], wsb_ref[...],
                           preferred_element_type=jnp.float32)
                 + bs_ref[...])
    o_ref[...] = _ln_prelu(fused_pre, g2_ref[...], be2_ref[...], a2_ref[0, 0])


def _s2s(h0_wide, h3_wide, batch, seq_feat, Wl_ih, bl_ih, Wl_hh, bl_hh, Wp,
         bp, g1, be1, a1, Ws, bs, g2, be2, a2):
    return pl.pallas_call(
        _s2s_body,
        out_shape=jax.ShapeDtypeStruct((NGRAPH, OUT), jnp.float32),
    )(h0_wide, h3_wide, batch.reshape(N, 1), seq_feat.reshape(1, OUT),
      Wl_ih.T, bl_ih.reshape(1, 8 * H), Wl_hh.T, bl_hh.reshape(1, 8 * H),
      Wp, bp.reshape(1, OUT), g1.reshape(1, OUT), be1.reshape(1, OUT),
      jnp.reshape(a1, (1, 1)), Ws[:OUT, :], Ws[OUT:, :], bs.reshape(1, OUT),
      g2.reshape(1, OUT), be2.reshape(1, OUT), jnp.reshape(a2, (1, 1)))


def kernel(x, edge_index, edge_attr, batch, seq_feat, W1, b1, We, be, root,
           bconv, W_ih, W_hh, b_ih, b_hh, Wl_ih, Wl_hh, bl_ih, bl_hh, Wp, bp,
           g1, be1, a1, Ws, bs, g2, be2, a2):
    n_extra = E_PAD - E
    pad_src = (jnp.arange(n_extra, dtype=jnp.int32) * 97) % N
    pad_dst = N + (jnp.arange(n_extra, dtype=jnp.int32) % (N_PAD - N))
    src3 = jnp.concatenate([edge_index[0], pad_src]).reshape(NW, NCH, CH)
    dst3 = jnp.concatenate([edge_index[1], pad_dst]).reshape(NW, NCH, CH)
    ea_pad = jnp.concatenate(
        [edge_attr, jnp.zeros((n_extra, EDGE_IN), jnp.float32)], axis=0)
    zeros_npad = jnp.zeros((N_PAD, W), jnp.float32)

    h0 = _node_mlp(x, W1, b1)
    WihT = W_ih.T
    WhhT = W_hh.T

    h = h0
    for _ in range(STEPS):
        hsrc = _sc_gather(h, src3)
        msg = _msg(ea_pad, hsrc, We, be)
        partials = _sc_scatter(msg, dst3, zeros_npad)
        h = _gru(partials, h, root, bconv, WihT, b_ih, WhhT, b_hh)

    return _s2s(h0, h, batch, seq_feat, Wl_ih, bl_ih, Wl_hh, bl_hh, Wp, bp,
                g1, be1, a1, Ws, bs, g2, be2, a2)


# speedup vs baseline: 3.1904x; 3.1904x over previous
import functools

import jax
import jax.numpy as jnp
from jax import lax
from jax.experimental import pallas as pl
from jax.experimental.pallas import tpu as pltpu
from jax.experimental.pallas import tpu_sc as plsc

N = 10000
E = 160000
NODE_IN = 128
EDGE_IN = 16
H = 32
W = 128
OUT = 128
STEPS = 3
S2S_STEPS = 3
NGRAPH = 256

NW = 32
CH = 128
E_PAD = 163840
EPW = E_PAD // NW
NCH = EPW // CH
N_PAD = 10240
ROWS_PER_TILE = N_PAD // 16


def _sc_gather_body(h_hbm, src_hbm, out_hbm, idx_all, rows, sem):
    c = lax.axis_index("c")
    s = lax.axis_index("s")
    wid = s * 2 + c
    pltpu.sync_copy(src_hbm.at[wid], idx_all)

    def chunk(j, carry):
        off = wid * EPW + j * CH
        pltpu.async_copy(h_hbm.at[idx_all.at[j]], rows, sem).wait()
        pltpu.sync_copy(rows, out_hbm.at[pl.ds(off, CH)])
        return carry

    lax.fori_loop(0, NCH, chunk, 0)


@functools.cache
def _sc_gather_call():
    mesh = plsc.VectorSubcoreMesh(core_axis_name="c", subcore_axis_name="s")
    return pl.kernel(
        _sc_gather_body,
        out_type=jax.ShapeDtypeStruct((E_PAD, W), jnp.float32),
        mesh=mesh,
        scratch_types=[
            pltpu.VMEM((NCH, CH), jnp.int32),
            pltpu.VMEM((CH, W), jnp.float32),
            pltpu.SemaphoreType.DMA,
        ],
    )


def _sc_gather(h_wide, src3):
    return _sc_gather_call()(h_wide, src3)


def _sc_scatter_body(msg_hbm, dst_hbm, zeros_hbm, out_hbm, idx_all, rows,
                     accum, sem):
    c = lax.axis_index("c")
    s = lax.axis_index("s")
    wid = s * 2 + c

    @pl.when(s == 0)
    def _():
        pltpu.sync_copy(zeros_hbm, accum)

    plsc.subcore_barrier()
    pltpu.sync_copy(dst_hbm.at[wid], idx_all)

    def chunk(j, carry):
        off = wid * EPW + j * CH
        pltpu.sync_copy(msg_hbm.at[pl.ds(off, CH)], rows)
        pltpu.sync_copy(rows, accum.at[idx_all.at[j]], add=True)
        return carry

    lax.fori_loop(0, NCH, chunk, 0)
    plsc.subcore_barrier()
    pltpu.sync_copy(accum.at[pl.ds(s * ROWS_PER_TILE, ROWS_PER_TILE)],
                    out_hbm.at[c, pl.ds(s * ROWS_PER_TILE, ROWS_PER_TILE)])


@functools.cache
def _sc_scatter_call():
    mesh = plsc.VectorSubcoreMesh(core_axis_name="c", subcore_axis_name="s")
    return pl.kernel(
        _sc_scatter_body,
        out_type=jax.ShapeDtypeStruct((2, N_PAD, W), jnp.float32),
        mesh=mesh,
        scratch_types=[
            pltpu.VMEM((NCH, CH), jnp.int32),
            pltpu.VMEM((CH, W), jnp.float32),
            pltpu.VMEM_SHARED((N_PAD, W), jnp.float32),
            pltpu.SemaphoreType.DMA,
        ],
    )


def _sc_scatter(msg_wide, dst3, zeros_npad):
    return _sc_scatter_call()(msg_wide, dst3, zeros_npad)


def _node_mlp_body(x_ref, w_ref, b_ref, o_ref):
    h = jnp.maximum(
        jnp.dot(x_ref[...], w_ref[...], preferred_element_type=jnp.float32)
        + b_ref[...], 0.0)
    o_ref[...] = jnp.concatenate(
        [h, jnp.zeros((N, W - H), jnp.float32)], axis=1)


def _node_mlp(x, W1, b1):
    return pl.pallas_call(
        _node_mlp_body,
        out_shape=jax.ShapeDtypeStruct((N, W), jnp.float32),
    )(x, W1, b1.reshape(1, H))


EB = 1024
N_EB = E_PAD // EB


def _msg_body(ea_ref, hs_ref, we_ref, be_ref, s_ref, o_ref):
    a = jnp.dot(ea_ref[...], we_ref[...], preferred_element_type=jnp.float32)
    ew = jnp.maximum(a + be_ref[...], 0.0)
    hsw = jnp.dot(hs_ref[:, 0:H], s_ref[...],
                  preferred_element_type=jnp.float32)
    m128 = ew[:, 0:W] * hsw[:, 0:W]
    for j in range(1, (H * H) // W):
        m128 = m128 + ew[:, j * W:(j + 1) * W] * hsw[:, j * W:(j + 1) * W]
    acc = (m128[:, 0:H] + m128[:, H:2 * H] + m128[:, 2 * H:3 * H]
           + m128[:, 3 * H:4 * H])
    o_ref[...] = jnp.concatenate(
        [acc, jnp.zeros((EB, W - H), jnp.float32)], axis=1)


def _msg(ea_pad, hsrc_wide, We, be, S_exp):
    return pl.pallas_call(
        _msg_body,
        grid=(N_EB,),
        in_specs=[
            pl.BlockSpec((EB, EDGE_IN), lambda i: (i, 0)),
            pl.BlockSpec((EB, W), lambda i: (i, 0)),
            pl.BlockSpec((EDGE_IN, H * H), lambda i: (0, 0)),
            pl.BlockSpec((1, H * H), lambda i: (0, 0)),
            pl.BlockSpec((H, H * H), lambda i: (0, 0)),
        ],
        out_specs=pl.BlockSpec((EB, W), lambda i: (i, 0)),
        out_shape=jax.ShapeDtypeStruct((E_PAD, W), jnp.float32),
        compiler_params=pltpu.CompilerParams(
            dimension_semantics=("arbitrary",)),
    )(ea_pad, hsrc_wide, We, be.reshape(1, H * H), S_exp)


def _gru_body(p_ref, h_ref, root_ref, bc_ref, wih_ref, bih_ref, whh_ref,
              bhh_ref, o_ref):
    h = h_ref[:, 0:H]
    aggr = p_ref[0, :N, 0:H] + p_ref[1, :N, 0:H]
    conv = aggr + jnp.dot(h, root_ref[...],
                          preferred_element_type=jnp.float32) + bc_ref[...]
    m = jnp.maximum(conv, 0.0)
    gi = jnp.dot(m, wih_ref[...], preferred_element_type=jnp.float32) \
        + bih_ref[...]
    gh = jnp.dot(h, whh_ref[...], preferred_element_type=jnp.float32) \
        + bhh_ref[...]
    r = jax.nn.sigmoid(gi[:, 0:H] + gh[:, 0:H])
    z = jax.nn.sigmoid(gi[:, H:2 * H] + gh[:, H:2 * H])
    n_ = jnp.tanh(gi[:, 2 * H:3 * H] + r * gh[:, 2 * H:3 * H])
    hn = (1.0 - z) * n_ + z * h
    o_ref[...] = jnp.concatenate(
        [hn, jnp.zeros((N, W - H), jnp.float32)], axis=1)


def _gru(partials, h_wide, root, bconv, WihT, b_ih, WhhT, b_hh):
    return pl.pallas_call(
        _gru_body,
        out_shape=jax.ShapeDtypeStruct((N, W), jnp.float32),
    )(partials, h_wide, root, bconv.reshape(1, H), WihT,
      b_ih.reshape(1, 3 * H), WhhT, b_hh.reshape(1, 3 * H))


def _ln_prelu(y, g, b, alpha):
    mu = jnp.mean(y, axis=-1, keepdims=True)
    var = jnp.mean((y - mu) ** 2, axis=-1, keepdims=True)
    yn = (y - mu) / jnp.sqrt(var + 1e-5) * g + b
    return jnp.where(yn >= 0, yn, alpha * yn)


S2S_CHK = 1000
S2S_NCHK = N // S2S_CHK


def _s2s_body(h0_ref, h3_ref, batch_ref, sf_ref, wlih_ref, blih_ref,
              wlhh_ref, blhh_ref, wp_ref, bp_ref, g1_ref, be1_ref, a1_ref,
              wst_ref, wsb_ref, bs_ref, g2_ref, be2_ref, a2_ref, o_ref):
    def chunk(i):
        sl = pl.ds(i * S2S_CHK, S2S_CHK)
        na_c = jnp.concatenate([h0_ref[sl, 0:H], h3_ref[sl, 0:H]], axis=1)
        gid = lax.broadcasted_iota(jnp.int32, (S2S_CHK, NGRAPH), 1)
        onehot = batch_ref[sl, :] == gid
        return na_c, onehot

    q_star = jnp.zeros((NGRAPH, 4 * H), jnp.float32)
    hl = jnp.zeros((NGRAPH, 2 * H), jnp.float32)
    cl = jnp.zeros((NGRAPH, 2 * H), jnp.float32)
    for _ in range(S2S_STEPS):
        gates = (jnp.dot(q_star, wlih_ref[...],
                         preferred_element_type=jnp.float32) + blih_ref[...]
                 + jnp.dot(hl, wlhh_ref[...],
                           preferred_element_type=jnp.float32)
                 + blhh_ref[...])
        ig = jax.nn.sigmoid(gates[:, 0:2 * H])
        fg = jax.nn.sigmoid(gates[:, 2 * H:4 * H])
        gg = jnp.tanh(gates[:, 4 * H:6 * H])
        og = jax.nn.sigmoid(gates[:, 6 * H:8 * H])
        cl = fg * cl + ig * gg
        hl = og * jnp.tanh(cl)
        q = hl

        def maxpass(i, emax):
            na_c, onehot = chunk(i)
            emat = lax.dot_general(na_c, q, (((1,), (1,)), ((), ())),
                                   preferred_element_type=jnp.float32)
            m = jnp.max(jnp.where(onehot, emat, -1e30), axis=0,
                        keepdims=True)
            return jnp.maximum(emax, m)

        emax = lax.fori_loop(0, S2S_NCHK, maxpass,
                             jnp.full((1, NGRAPH), -1e30, jnp.float32))
        emax = jnp.where(emax > -1e29, emax, 0.0)

        def sumpass(i, carry):
            denom8, U = carry
            na_c, onehot = chunk(i)
            onef = onehot.astype(jnp.float32)
            emat = lax.dot_general(na_c, q, (((1,), (1,)), ((), ())),
                                   preferred_element_type=jnp.float32)
            e = jnp.sum(jnp.where(onehot, emat, 0.0), axis=1, keepdims=True)
            emaxb = jnp.sum(onef * emax, axis=1, keepdims=True)
            ex = jnp.exp(e - emaxb)
            w = onef * ex
            d8 = lax.dot_general(w, jnp.ones((S2S_CHK, 8), jnp.float32),
                                 (((0,), (0,)), ((), ())),
                                 preferred_element_type=jnp.float32)
            dU = lax.dot_general(w, na_c, (((0,), (0,)), ((), ())),
                                 preferred_element_type=jnp.float32)
            return denom8 + d8, U + dU

        denom8, U = lax.fori_loop(
            0, S2S_NCHK, sumpass,
            (jnp.zeros((NGRAPH, 8), jnp.float32),
             jnp.zeros((NGRAPH, 2 * H), jnp.float32)))
        r_ = U / (denom8[:, 0:1] + 1e-16)
        q_star = jnp.concatenate([q, r_], axis=1)

    go = _ln_prelu(jnp.dot(q_star, wp_ref[...],
                           preferred_element_type=jnp.float32) + bp_ref[...],
                   g1_ref[...], be1_ref[...], a1_ref[0, 0])
    fused_pre = (jnp.dot(go, wst_ref[...],
                         preferred_element_type=jnp.float32)
                 + jnp.dot(sf_ref[...], wsb_ref[...],
                           preferred_element_type=jnp.float32)
                 + bs_ref[...])
    o_ref[...] = _ln_prelu(fused_pre, g2_ref[...], be2_ref[...], a2_ref[0, 0])


def _s2s(h0_wide, h3_wide, batch, seq_feat, Wl_ih, bl_ih, Wl_hh, bl_hh, Wp,
         bp, g1, be1, a1, Ws, bs, g2, be2, a2):
    return pl.pallas_call(
        _s2s_body,
        out_shape=jax.ShapeDtypeStruct((NGRAPH, OUT), jnp.float32),
    )(h0_wide, h3_wide, batch.reshape(N, 1), seq_feat.reshape(1, OUT),
      Wl_ih.T, bl_ih.reshape(1, 8 * H), Wl_hh.T, bl_hh.reshape(1, 8 * H),
      Wp, bp.reshape(1, OUT), g1.reshape(1, OUT), be1.reshape(1, OUT),
      jnp.reshape(a1, (1, 1)), Ws[:OUT, :], Ws[OUT:, :], bs.reshape(1, OUT),
      g2.reshape(1, OUT), be2.reshape(1, OUT), jnp.reshape(a2, (1, 1)))


def kernel(x, edge_index, edge_attr, batch, seq_feat, W1, b1, We, be, root,
           bconv, W_ih, W_hh, b_ih, b_hh, Wl_ih, Wl_hh, bl_ih, bl_hh, Wp, bp,
           g1, be1, a1, Ws, bs, g2, be2, a2):
    n_extra = E_PAD - E
    pad_src = (jnp.arange(n_extra, dtype=jnp.int32) * 97) % N
    pad_dst = N + (jnp.arange(n_extra, dtype=jnp.int32) % (N_PAD - N))
    src3 = jnp.concatenate([edge_index[0], pad_src]).reshape(NW, NCH, CH)
    dst3 = jnp.concatenate([edge_index[1], pad_dst]).reshape(NW, NCH, CH)
    ea_pad = jnp.concatenate(
        [edge_attr, jnp.zeros((n_extra, EDGE_IN), jnp.float32)], axis=0)
    zeros_npad = jnp.zeros((N_PAD, W), jnp.float32)
    S_exp = (jnp.arange(H * H, dtype=jnp.int32)[None, :] // H
             == jnp.arange(H, dtype=jnp.int32)[:, None]).astype(jnp.float32)

    h0 = _node_mlp(x, W1, b1)
    WihT = W_ih.T
    WhhT = W_hh.T

    h = h0
    for _ in range(STEPS):
        hsrc = _sc_gather(h, src3)
        msg = _msg(ea_pad, hsrc, We, be, S_exp)
        partials = _sc_scatter(msg, dst3, zeros_npad)
        h = _gru(partials, h, root, bconv, WihT, b_ih, WhhT, b_hh)

    return _s2s(h0, h, batch, seq_feat, Wl_ih, bl_ih, Wl_hh, bl_hh, Wp, bp,
                g1, be1, a1, Ws, bs, g2, be2, a2)


# speedup vs baseline: 3.3375x; 1.0461x over previous
import functools

import jax
import jax.numpy as jnp
from jax import lax
from jax.experimental import pallas as pl
from jax.experimental.pallas import tpu as pltpu
from jax.experimental.pallas import tpu_sc as plsc

N = 10000
E = 160000
NODE_IN = 128
EDGE_IN = 16
H = 32
W = 128
OUT = 128
STEPS = 3
S2S_STEPS = 3
NGRAPH = 256

NW = 32
CH = 128
E_PAD = 163840
EPW = E_PAD // NW
NCH = EPW // CH
N_PAD = 10240
ROWS_PER_TILE = N_PAD // 16


GRP = 4
NGRP = NCH // GRP


def _sc_gather_body(h_hbm, src_hbm, out_hbm, idx_all, r0, r1, r2, r3, sem):
    c = lax.axis_index("c")
    s = lax.axis_index("s")
    wid = s * 2 + c
    pltpu.sync_copy(src_hbm.at[wid], idx_all)
    bufs = [r0, r1, r2, r3]

    def grp(i, carry):
        handles = [
            pltpu.async_copy(h_hbm.at[idx_all.at[i * GRP + b]], bufs[b], sem)
            for b in range(GRP)
        ]
        for b, hdl in enumerate(handles):
            hdl.wait()
            off = wid * EPW + (i * GRP + b) * CH
            pltpu.sync_copy(bufs[b], out_hbm.at[pl.ds(off, CH)])
        return carry

    lax.fori_loop(0, NGRP, grp, 0)


@functools.cache
def _sc_gather_call():
    mesh = plsc.VectorSubcoreMesh(core_axis_name="c", subcore_axis_name="s")
    return pl.kernel(
        _sc_gather_body,
        out_type=jax.ShapeDtypeStruct((E_PAD, W), jnp.float32),
        mesh=mesh,
        scratch_types=[
            pltpu.VMEM((NCH, CH), jnp.int32),
            pltpu.VMEM((CH, W), jnp.float32),
            pltpu.VMEM((CH, W), jnp.float32),
            pltpu.VMEM((CH, W), jnp.float32),
            pltpu.VMEM((CH, W), jnp.float32),
            pltpu.SemaphoreType.DMA,
        ],
    )


def _sc_gather(h_wide, src3):
    return _sc_gather_call()(h_wide, src3)


def _sc_scatter_body(msg_hbm, dst_hbm, zeros_hbm, out_hbm, idx_all, rows,
                     accum, sem):
    c = lax.axis_index("c")
    s = lax.axis_index("s")
    wid = s * 2 + c

    @pl.when(s == 0)
    def _():
        pltpu.sync_copy(zeros_hbm, accum)

    plsc.subcore_barrier()
    pltpu.sync_copy(dst_hbm.at[wid], idx_all)

    def chunk(j, carry):
        off = wid * EPW + j * CH
        pltpu.sync_copy(msg_hbm.at[pl.ds(off, CH)], rows)
        pltpu.sync_copy(rows, accum.at[idx_all.at[j]], add=True)
        return carry

    lax.fori_loop(0, NCH, chunk, 0)
    plsc.subcore_barrier()
    pltpu.sync_copy(accum.at[pl.ds(s * ROWS_PER_TILE, ROWS_PER_TILE)],
                    out_hbm.at[c, pl.ds(s * ROWS_PER_TILE, ROWS_PER_TILE)])


@functools.cache
def _sc_scatter_call():
    mesh = plsc.VectorSubcoreMesh(core_axis_name="c", subcore_axis_name="s")
    return pl.kernel(
        _sc_scatter_body,
        out_type=jax.ShapeDtypeStruct((2, N_PAD, W), jnp.float32),
        mesh=mesh,
        scratch_types=[
            pltpu.VMEM((NCH, CH), jnp.int32),
            pltpu.VMEM((CH, W), jnp.float32),
            pltpu.VMEM_SHARED((N_PAD, W), jnp.float32),
            pltpu.SemaphoreType.DMA,
        ],
    )


def _sc_scatter(msg_wide, dst3, zeros_npad):
    return _sc_scatter_call()(msg_wide, dst3, zeros_npad)


def _node_mlp_body(x_ref, w_ref, b_ref, o_ref):
    h = jnp.maximum(
        jnp.dot(x_ref[...], w_ref[...], preferred_element_type=jnp.float32)
        + b_ref[...], 0.0)
    o_ref[...] = jnp.concatenate(
        [h, jnp.zeros((N, W - H), jnp.float32)], axis=1)


def _node_mlp(x, W1, b1):
    return pl.pallas_call(
        _node_mlp_body,
        out_shape=jax.ShapeDtypeStruct((N, W), jnp.float32),
    )(x, W1, b1.reshape(1, H))


EB = 1024
N_EB = E_PAD // EB


def _msg_body(ea_ref, hs_ref, we_ref, be_ref, s_ref, o_ref):
    a = jnp.dot(ea_ref[...], we_ref[...], preferred_element_type=jnp.float32)
    ew = jnp.maximum(a + be_ref[...], 0.0)
    hsw = jnp.dot(hs_ref[:, 0:H], s_ref[...],
                  preferred_element_type=jnp.float32)
    m128 = ew[:, 0:W] * hsw[:, 0:W]
    for j in range(1, (H * H) // W):
        m128 = m128 + ew[:, j * W:(j + 1) * W] * hsw[:, j * W:(j + 1) * W]
    acc = (m128[:, 0:H] + m128[:, H:2 * H] + m128[:, 2 * H:3 * H]
           + m128[:, 3 * H:4 * H])
    o_ref[...] = jnp.concatenate(
        [acc, jnp.zeros((EB, W - H), jnp.float32)], axis=1)


def _msg(ea_pad, hsrc_wide, We, be, S_exp):
    return pl.pallas_call(
        _msg_body,
        grid=(N_EB,),
        in_specs=[
            pl.BlockSpec((EB, EDGE_IN), lambda i: (i, 0)),
            pl.BlockSpec((EB, W), lambda i: (i, 0)),
            pl.BlockSpec((EDGE_IN, H * H), lambda i: (0, 0)),
            pl.BlockSpec((1, H * H), lambda i: (0, 0)),
            pl.BlockSpec((H, H * H), lambda i: (0, 0)),
        ],
        out_specs=pl.BlockSpec((EB, W), lambda i: (i, 0)),
        out_shape=jax.ShapeDtypeStruct((E_PAD, W), jnp.float32),
        compiler_params=pltpu.CompilerParams(
            dimension_semantics=("arbitrary",)),
    )(ea_pad, hsrc_wide, We, be.reshape(1, H * H), S_exp)


def _gru_body(p_ref, h_ref, root_ref, bc_ref, wih_ref, bih_ref, whh_ref,
              bhh_ref, o_ref):
    h = h_ref[:, 0:H]
    aggr = p_ref[0, :N, 0:H] + p_ref[1, :N, 0:H]
    conv = aggr + jnp.dot(h, root_ref[...],
                          preferred_element_type=jnp.float32) + bc_ref[...]
    m = jnp.maximum(conv, 0.0)
    gi = jnp.dot(m, wih_ref[...], preferred_element_type=jnp.float32) \
        + bih_ref[...]
    gh = jnp.dot(h, whh_ref[...], preferred_element_type=jnp.float32) \
        + bhh_ref[...]
    r = jax.nn.sigmoid(gi[:, 0:H] + gh[:, 0:H])
    z = jax.nn.sigmoid(gi[:, H:2 * H] + gh[:, H:2 * H])
    n_ = jnp.tanh(gi[:, 2 * H:3 * H] + r * gh[:, 2 * H:3 * H])
    hn = (1.0 - z) * n_ + z * h
    o_ref[...] = jnp.concatenate(
        [hn, jnp.zeros((N, W - H), jnp.float32)], axis=1)


def _gru(partials, h_wide, root, bconv, WihT, b_ih, WhhT, b_hh):
    return pl.pallas_call(
        _gru_body,
        out_shape=jax.ShapeDtypeStruct((N, W), jnp.float32),
    )(partials, h_wide, root, bconv.reshape(1, H), WihT,
      b_ih.reshape(1, 3 * H), WhhT, b_hh.reshape(1, 3 * H))


def _ln_prelu(y, g, b, alpha):
    mu = jnp.mean(y, axis=-1, keepdims=True)
    var = jnp.mean((y - mu) ** 2, axis=-1, keepdims=True)
    yn = (y - mu) / jnp.sqrt(var + 1e-5) * g + b
    return jnp.where(yn >= 0, yn, alpha * yn)


S2S_CHK = 1000
S2S_NCHK = N // S2S_CHK


def _s2s_body(h0_ref, h3_ref, batch_ref, sf_ref, wlih_ref, blih_ref,
              wlhh_ref, blhh_ref, wp_ref, bp_ref, g1_ref, be1_ref, a1_ref,
              wst_ref, wsb_ref, bs_ref, g2_ref, be2_ref, a2_ref, o_ref):
    def chunk(i):
        sl = pl.ds(i * S2S_CHK, S2S_CHK)
        na_c = jnp.concatenate([h0_ref[sl, 0:H], h3_ref[sl, 0:H]], axis=1)
        gid = lax.broadcasted_iota(jnp.int32, (S2S_CHK, NGRAPH), 1)
        onehot = batch_ref[sl, :] == gid
        return na_c, onehot

    q_star = jnp.zeros((NGRAPH, 4 * H), jnp.float32)
    hl = jnp.zeros((NGRAPH, 2 * H), jnp.float32)
    cl = jnp.zeros((NGRAPH, 2 * H), jnp.float32)
    for _ in range(S2S_STEPS):
        gates = (jnp.dot(q_star, wlih_ref[...],
                         preferred_element_type=jnp.float32) + blih_ref[...]
                 + jnp.dot(hl, wlhh_ref[...],
                           preferred_element_type=jnp.float32)
                 + blhh_ref[...])
        ig = jax.nn.sigmoid(gates[:, 0:2 * H])
        fg = jax.nn.sigmoid(gates[:, 2 * H:4 * H])
        gg = jnp.tanh(gates[:, 4 * H:6 * H])
        og = jax.nn.sigmoid(gates[:, 6 * H:8 * H])
        cl = fg * cl + ig * gg
        hl = og * jnp.tanh(cl)
        q = hl

        def maxpass(i, emax):
            na_c, onehot = chunk(i)
            emat = lax.dot_general(na_c, q, (((1,), (1,)), ((), ())),
                                   preferred_element_type=jnp.float32)
            m = jnp.max(jnp.where(onehot, emat, -1e30), axis=0,
                        keepdims=True)
            return jnp.maximum(emax, m)

        emax = lax.fori_loop(0, S2S_NCHK, maxpass,
                             jnp.full((1, NGRAPH), -1e30, jnp.float32))
        emax = jnp.where(emax > -1e29, emax, 0.0)

        def sumpass(i, carry):
            denom8, U = carry
            na_c, onehot = chunk(i)
            onef = onehot.astype(jnp.float32)
            emat = lax.dot_general(na_c, q, (((1,), (1,)), ((), ())),
                                   preferred_element_type=jnp.float32)
            e = jnp.sum(jnp.where(onehot, emat, 0.0), axis=1, keepdims=True)
            emaxb = jnp.sum(onef * emax, axis=1, keepdims=True)
            ex = jnp.exp(e - emaxb)
            w = onef * ex
            d8 = lax.dot_general(w, jnp.ones((S2S_CHK, 8), jnp.float32),
                                 (((0,), (0,)), ((), ())),
                                 preferred_element_type=jnp.float32)
            dU = lax.dot_general(w, na_c, (((0,), (0,)), ((), ())),
                                 preferred_element_type=jnp.float32)
            return denom8 + d8, U + dU

        denom8, U = lax.fori_loop(
            0, S2S_NCHK, sumpass,
            (jnp.zeros((NGRAPH, 8), jnp.float32),
             jnp.zeros((NGRAPH, 2 * H), jnp.float32)))
        r_ = U / (denom8[:, 0:1] + 1e-16)
        q_star = jnp.concatenate([q, r_], axis=1)

    go = _ln_prelu(jnp.dot(q_star, wp_ref[...],
                           preferred_element_type=jnp.float32) + bp_ref[...],
                   g1_ref[...], be1_ref[...], a1_ref[0, 0])
    fused_pre = (jnp.dot(go, wst_ref[...],
                         preferred_element_type=jnp.float32)
                 + jnp.dot(sf_ref[...], wsb_ref[...],
                           preferred_element_type=jnp.float32)
                 + bs_ref[...])
    o_ref[...] = _ln_prelu(fused_pre, g2_ref[...], be2_ref[...], a2_ref[0, 0])


def _s2s(h0_wide, h3_wide, batch, seq_feat, Wl_ih, bl_ih, Wl_hh, bl_hh, Wp,
         bp, g1, be1, a1, Ws, bs, g2, be2, a2):
    return pl.pallas_call(
        _s2s_body,
        out_shape=jax.ShapeDtypeStruct((NGRAPH, OUT), jnp.float32),
    )(h0_wide, h3_wide, batch.reshape(N, 1), seq_feat.reshape(1, OUT),
      Wl_ih.T, bl_ih.reshape(1, 8 * H), Wl_hh.T, bl_hh.reshape(1, 8 * H),
      Wp, bp.reshape(1, OUT), g1.reshape(1, OUT), be1.reshape(1, OUT),
      jnp.reshape(a1, (1, 1)), Ws[:OUT, :], Ws[OUT:, :], bs.reshape(1, OUT),
      g2.reshape(1, OUT), be2.reshape(1, OUT), jnp.reshape(a2, (1, 1)))


def kernel(x, edge_index, edge_attr, batch, seq_feat, W1, b1, We, be, root,
           bconv, W_ih, W_hh, b_ih, b_hh, Wl_ih, Wl_hh, bl_ih, bl_hh, Wp, bp,
           g1, be1, a1, Ws, bs, g2, be2, a2):
    n_extra = E_PAD - E
    pad_src = (jnp.arange(n_extra, dtype=jnp.int32) * 97) % N
    pad_dst = N + (jnp.arange(n_extra, dtype=jnp.int32) % (N_PAD - N))
    src3 = jnp.concatenate([edge_index[0], pad_src]).reshape(NW, NCH, CH)
    dst3 = jnp.concatenate([edge_index[1], pad_dst]).reshape(NW, NCH, CH)
    ea_pad = jnp.concatenate(
        [edge_attr, jnp.zeros((n_extra, EDGE_IN), jnp.float32)], axis=0)
    zeros_npad = jnp.zeros((N_PAD, W), jnp.float32)
    S_exp = (jnp.arange(H * H, dtype=jnp.int32)[None, :] // H
             == jnp.arange(H, dtype=jnp.int32)[:, None]).astype(jnp.float32)

    h0 = _node_mlp(x, W1, b1)
    WihT = W_ih.T
    WhhT = W_hh.T

    h = h0
    for _ in range(STEPS):
        hsrc = _sc_gather(h, src3)
        msg = _msg(ea_pad, hsrc, We, be, S_exp)
        partials = _sc_scatter(msg, dst3, zeros_npad)
        h = _gru(partials, h, root, bconv, WihT, b_ih, WhhT, b_hh)

    return _s2s(h0, h, batch, seq_feat, Wl_ih, bl_ih, Wl_hh, bl_hh, Wp, bp,
                g1, be1, a1, Ws, bs, g2, be2, a2)
